# Initial kernel scaffold; baseline (speedup 1.0000x reference)
import jax
import jax.numpy as jnp
from jax.experimental import pallas as pl


def kernel(x, a_2, b_2, w):
    raise NotImplementedError("write your pallas kernel here")


# speedup vs baseline: 2.3048x; 2.3048x over previous
import functools
import math

import jax
import jax.numpy as jnp
from jax.experimental import pallas as pl
from jax.experimental.pallas import tpu as pltpu

_BLOCK_ROWS = 512


def _fused_ln_ff_residual_kernel(x_ref, g_ref, b_ref, w_ref, o_ref, *, eps: float):
    x = x_ref[...]
    f = x.shape[-1]
    mean = jnp.sum(x, axis=-1, keepdims=True) * jnp.float32(1.0 / f)
    xc = x - mean
    var = jnp.sum(xc * xc, axis=-1, keepdims=True) * jnp.float32(1.0 / (f - 1))
    inv = pl.reciprocal(jnp.sqrt(var) + jnp.float32(eps), approx=False)
    h = xc * inv * g_ref[...] + b_ref[...]
    y = jnp.dot(h.astype(jnp.bfloat16), w_ref[...],
                preferred_element_type=jnp.float32)
    o_ref[...] = x + y


def kernel(x, a_2, b_2, w, eps: float = 1e-6):
    orig_shape = x.shape
    features = orig_shape[-1]
    rows = math.prod(orig_shape[:-1])
    x2 = x.reshape(rows, features)
    g2 = a_2.astype(jnp.float32).reshape(1, features)
    b2 = b_2.astype(jnp.float32).reshape(1, features)
    w_bf16 = w.astype(jnp.bfloat16)

    block_rows = min(_BLOCK_ROWS, rows)
    grid = (pl.cdiv(rows, block_rows),)
    row_spec = pl.BlockSpec((block_rows, features), lambda i: (i, 0))

    out = pl.pallas_call(
        functools.partial(_fused_ln_ff_residual_kernel, eps=eps),
        out_shape=jax.ShapeDtypeStruct((rows, features), x.dtype),
        grid=grid,
        in_specs=[
            row_spec,
            pl.BlockSpec((1, features), lambda i: (0, 0)),
            pl.BlockSpec((1, features), lambda i: (0, 0)),
            pl.BlockSpec((features, features), lambda i: (0, 0)),
        ],
        out_specs=row_spec,
        compiler_params=pltpu.CompilerParams(
            dimension_semantics=("parallel",),
            vmem_limit_bytes=48 * 1024 * 1024,
        ),
    )(x2, g2, b2, w_bf16)

    return out.reshape(orig_shape)
